# Initial kernel scaffold; baseline (speedup 1.0000x reference)
import jax
import jax.numpy as jnp
from jax.experimental import pallas as pl


def kernel(sr_data, tg_data, h_list_sr, h_list_tg, t_list_sr, t_list_tg, r_list_sr, r_list_tg, edge_index_sr, edge_index_tg, ent_emb_sr, ent_emb_tg, rel_emb_sr, rel_emb_tg, gat_W, gat_a_src, gat_a_dst):
    raise NotImplementedError("write your pallas kernel here")


# speedup vs baseline: 9.4153x; 9.4153x over previous
import functools

import jax
import jax.numpy as jnp
from jax import lax
from jax.experimental import pallas as pl
from jax.experimental.pallas import tpu as pltpu
from jax.experimental.pallas import tpu_sc as plsc

N = 10000
NPAD = 10112
DIM = 128
DH = 64
NHEADS = 2
NLAYER = 2
E = 320000
T = 100000
BATCH = 4096
ALPHA = 0.2

NTILES = 32
CHW = 128
NCH = 80
EPT = NCH * CHW
EPAD = NTILES * EPT
NPT = NPAD // 16

NFC = T // CHW
TREM = T - NFC * CHW
TITER = (NFC + NTILES - 1) // NTILES + 1
TROWS = NFC + 1
BPT = BATCH // NTILES

BM = 632


def _proj_body(x_ref, w_ref, a_ref, wh_ref, ev_ref):
    wh = jnp.dot(x_ref[...], w_ref[...], preferred_element_type=jnp.float32)
    wh_ref[...] = wh
    ev_ref[...] = jnp.dot(wh, a_ref[...], preferred_element_type=jnp.float32)


def _proj(x, wc, ac):
    return pl.pallas_call(
        _proj_body,
        grid=(NPAD // BM,),
        in_specs=[
            pl.BlockSpec((BM, DIM), lambda i: (i, 0)),
            pl.BlockSpec((DIM, DIM), lambda i: (0, 0)),
            pl.BlockSpec((DIM, DIM), lambda i: (0, 0)),
        ],
        out_specs=[
            pl.BlockSpec((BM, DIM), lambda i: (i, 0)),
            pl.BlockSpec((BM, DIM), lambda i: (i, 0)),
        ],
        out_shape=[
            jax.ShapeDtypeStruct((NPAD, DIM), jnp.float32),
            jax.ShapeDtypeStruct((NPAD, DIM), jnp.float32),
        ],
    )(x, wc, ac)


def _normalize(r00, r01, r10, r11, d00, d01, d10, d11):
    raw0 = r00[...] + r01[...]
    raw1 = r10[...] + r11[...]
    den0 = d00[...] + d01[...]
    den1 = d10[...] + d11[...]
    rec0 = 1.0 / (den0[:, 0:1] + 1e-16)
    rec1 = 1.0 / (den1[:, 0:1] + 1e-16)
    return jnp.concatenate(
        [raw0 * jnp.broadcast_to(rec0, (BM, DH)),
         raw1 * jnp.broadcast_to(rec1, (BM, DH))], axis=1)


_NORM_SPECS = (
    [pl.BlockSpec((BM, DH), lambda i: (i, 0))] * 4
    + [pl.BlockSpec((BM, 16), lambda i: (i, 0))] * 4
)


def _combine_project_body(r00, r01, r10, r11, d00, d01, d10, d11,
                          w_ref, a_ref, wh_ref, ev_ref):
    h = _normalize(r00, r01, r10, r11, d00, d01, d10, d11)
    h = jnp.where(h > 0, h, jnp.exp(h) - 1.0)
    wh = jnp.dot(h, w_ref[...], preferred_element_type=jnp.float32)
    wh_ref[...] = wh
    ev_ref[...] = jnp.dot(wh, a_ref[...], preferred_element_type=jnp.float32)


def _combine_project(raw0, den0, raw1, den1, wc, ac):
    return pl.pallas_call(
        _combine_project_body,
        grid=(NPAD // BM,),
        in_specs=_NORM_SPECS + [
            pl.BlockSpec((DIM, DIM), lambda i: (0, 0)),
            pl.BlockSpec((DIM, DIM), lambda i: (0, 0)),
        ],
        out_specs=[
            pl.BlockSpec((BM, DIM), lambda i: (i, 0)),
            pl.BlockSpec((BM, DIM), lambda i: (i, 0)),
        ],
        out_shape=[
            jax.ShapeDtypeStruct((NPAD, DIM), jnp.float32),
            jax.ShapeDtypeStruct((NPAD, DIM), jnp.float32),
        ],
    )(raw0[0], raw0[1], raw1[0], raw1[1],
      den0[0], den0[1], den1[0], den1[1], wc, ac)


def _combine_final_body(r00, r01, r10, r11, d00, d01, d10, d11, o_ref):
    o_ref[...] = _normalize(r00, r01, r10, r11, d00, d01, d10, d11)


def _combine_final(raw0, den0, raw1, den1):
    return pl.pallas_call(
        _combine_final_body,
        grid=(NPAD // BM,),
        in_specs=_NORM_SPECS,
        out_specs=pl.BlockSpec((BM, DIM), lambda i: (i, 0)),
        out_shape=jax.ShapeDtypeStruct((NPAD, DIM), jnp.float32),
    )(raw0[0], raw0[1], raw1[0], raw1[1],
      den0[0], den0[1], den1[0], den1[1])


def _edge_sc(wh, ev16, srcp, dstp, z64, z16):
    mesh = plsc.VectorSubcoreMesh(core_axis_name="c", subcore_axis_name="s")

    @functools.partial(
        pl.kernel,
        out_type=(
            jax.ShapeDtypeStruct((2, 2, NPAD, DH), jnp.float32),
            jax.ShapeDtypeStruct((2, 2, NPAD, 16), jnp.float32),
        ),
        mesh=mesh,
        scratch_types=[
            pltpu.VMEM((CHW, 16), jnp.float32),
            pltpu.VMEM((CHW, 16), jnp.float32),
            pltpu.VMEM((NCH, CHW), jnp.int32),
            pltpu.VMEM((NCH, CHW), jnp.int32),
            pltpu.VMEM((CHW, DIM), jnp.float32),
            pltpu.VMEM((CHW, DH), jnp.float32),
            pltpu.VMEM((CHW, 16), jnp.float32),
            pltpu.VMEM_SHARED((NPAD, DH), jnp.float32),
            pltpu.VMEM_SHARED((NPAD, 16), jnp.float32),
            pltpu.VMEM_SHARED((NPAD, 16), jnp.float32),
        ],
        compiler_params=pltpu.CompilerParams(use_tc_tiling_on_sc=False),
    )
    def k(wh_hbm, ev_hbm, src_hbm, dst_hbm, z64_hbm, z16_hbm,
          raw_out, den_out, sbuf, dbuf, sidx, didx, rows, hrows, stag,
          spraw, spden, spev):
        cid = lax.axis_index("c")
        sid = lax.axis_index("s")
        wid = cid * 16 + sid
        r0 = sid * NPT
        pltpu.sync_copy(ev_hbm.at[pl.ds(r0, NPT)], spev.at[pl.ds(r0, NPT)])
        pltpu.sync_copy(src_hbm.at[pl.ds(wid * NCH, NCH)], sidx)
        pltpu.sync_copy(dst_hbm.at[pl.ds(wid * NCH, NCH)], didx)

        base_eid = wid * EPT
        iota = lax.iota(jnp.int32, 16)
        m_lane0 = iota == 0

        for head in range(NHEADS):
            pltpu.sync_copy(z64_hbm.at[pl.ds(r0, NPT)],
                            spraw.at[pl.ds(r0, NPT)])
            pltpu.sync_copy(z16_hbm.at[pl.ds(r0, NPT)],
                            spden.at[pl.ds(r0, NPT)])
            plsc.subcore_barrier()

            def chunk(j, carry, head=head):
                pltpu.sync_copy(wh_hbm.at[sidx.at[j]], rows)
                pltpu.sync_copy(spev.at[sidx.at[j]], sbuf)
                pltpu.sync_copy(spev.at[didx.at[j]], dbuf)
                base_j = base_eid + j * CHW

                def edge(kk, c):
                    vs = sbuf[kk, pl.ds(0, 16)]
                    vd = dbuf[kk, pl.ds(0, 16)]
                    e = vs[2 * head] + vd[2 * head + 1]
                    e = jnp.maximum(e, ALPHA * e)
                    ev = jnp.where(m_lane0, e, 0.0)
                    exv = jnp.exp(ev)
                    exv = jnp.where(m_lane0, exv, 0.0)
                    valid = (base_j + kk) < E
                    exv = jnp.where(valid, exv, 0.0)
                    stag[kk, pl.ds(0, 16)] = exv
                    x = exv[0]
                    for cc in range(DH // 16):
                        src_sl = pl.ds(head * DH + cc * 16, 16)
                        hrows[kk, pl.ds(cc * 16, 16)] = rows[kk, src_sl] * x
                    return c

                lax.fori_loop(0, CHW, edge, 0)
                pltpu.sync_copy(hrows, spraw.at[didx.at[j]], add=True)
                pltpu.sync_copy(stag, spden.at[didx.at[j]], add=True)
                return carry

            lax.fori_loop(0, NCH, chunk, 0)
            plsc.subcore_barrier()
            pltpu.sync_copy(spraw.at[pl.ds(r0, NPT)],
                            raw_out.at[head, cid, pl.ds(r0, NPT)])
            pltpu.sync_copy(spden.at[pl.ds(r0, NPT)],
                            den_out.at[head, cid, pl.ds(r0, NPT)])

    return k(wh, ev16, srcp, dstp, z64, z16)


def _post_sc(out_sr, out_tg, rel_sr, rel_tg,
             hs, ts, rs, ht, tt, rt, srd, tgd):
    mesh = plsc.VectorSubcoreMesh(core_axis_name="c", subcore_axis_name="s")

    @functools.partial(
        pl.kernel,
        out_type=(
            jax.ShapeDtypeStruct((2 * T, DIM), jnp.float32),
            jax.ShapeDtypeStruct((BATCH, DIM), jnp.float32),
            jax.ShapeDtypeStruct((BATCH, DIM), jnp.float32),
        ),
        mesh=mesh,
        scratch_types=[
            pltpu.VMEM((1, CHW), jnp.int32),
            pltpu.VMEM((1, CHW), jnp.int32),
            pltpu.VMEM((1, CHW), jnp.int32),
            pltpu.VMEM((CHW, DIM), jnp.float32),
            pltpu.VMEM((CHW, DIM), jnp.float32),
            pltpu.VMEM((CHW, DIM), jnp.float32),
            pltpu.VMEM((1, BPT), jnp.int32),
            pltpu.VMEM((BPT, DIM), jnp.float32),
        ],
    )
    def k(osr_hbm, otg_hbm, rsr_hbm, rtg_hbm,
          hs_hbm, ts_hbm, rs_hbm, ht_hbm, tt_hbm, rt_hbm, srd_hbm, tgd_hbm,
          tr_out, selsr_out, seltg_out,
          hidx, tidx, ridx, hbuf, tbuf, rbuf, bidx, selbuf):
        cid = lax.axis_index("c")
        sid = lax.axis_index("s")
        wid = cid * 16 + sid

        def combine(nrows):
            def row(kk, cc):
                for c8 in range(DIM // 16):
                    sl = pl.ds(c8 * 16, 16)
                    hbuf[kk, sl] = hbuf[kk, sl] + rbuf[kk, sl] - tbuf[kk, sl]
                return cc
            lax.fori_loop(0, nrows, row, 0)

        for graph in range(2):
            ent = osr_hbm if graph == 0 else otg_hbm
            rel = rsr_hbm if graph == 0 else rtg_hbm
            hl = hs_hbm if graph == 0 else ht_hbm
            tl = ts_hbm if graph == 0 else tt_hbm
            rl = rs_hbm if graph == 0 else rt_hbm
            obase = graph * T

            def tchunk(i, c, ent=ent, rel=rel, hl=hl, tl=tl, rl=rl,
                       obase=obase):
                cix = wid + i * NTILES

                @pl.when(cix < NFC)
                def _():
                    pltpu.sync_copy(hl.at[cix], hidx)
                    pltpu.sync_copy(tl.at[cix], tidx)
                    pltpu.sync_copy(rl.at[cix], ridx)
                    pltpu.sync_copy(ent.at[hidx.at[0]], hbuf)
                    pltpu.sync_copy(ent.at[tidx.at[0]], tbuf)
                    pltpu.sync_copy(rel.at[ridx.at[0]], rbuf)
                    combine(CHW)
                    pltpu.sync_copy(
                        hbuf, tr_out.at[pl.ds(obase + cix * CHW, CHW)])
                return c

            lax.fori_loop(0, TITER, tchunk, 0)

            @pl.when(wid == NFC % NTILES)
            def _(ent=ent, rel=rel, hl=hl, tl=tl, rl=rl, obase=obase):
                pltpu.sync_copy(hl.at[NFC], hidx)
                pltpu.sync_copy(tl.at[NFC], tidx)
                pltpu.sync_copy(rl.at[NFC], ridx)
                pltpu.sync_copy(ent.at[hidx.at[0, pl.ds(0, TREM)]],
                                hbuf.at[pl.ds(0, TREM)])
                pltpu.sync_copy(ent.at[tidx.at[0, pl.ds(0, TREM)]],
                                tbuf.at[pl.ds(0, TREM)])
                pltpu.sync_copy(rel.at[ridx.at[0, pl.ds(0, TREM)]],
                                rbuf.at[pl.ds(0, TREM)])
                combine(TREM)
                pltpu.sync_copy(hbuf.at[pl.ds(0, TREM)],
                                tr_out.at[pl.ds(obase + NFC * CHW, TREM)])

        pltpu.sync_copy(srd_hbm.at[wid], bidx)
        pltpu.sync_copy(osr_hbm.at[bidx.at[0]], selbuf)
        pltpu.sync_copy(selbuf, selsr_out.at[pl.ds(wid * BPT, BPT)])
        pltpu.sync_copy(tgd_hbm.at[wid], bidx)
        pltpu.sync_copy(otg_hbm.at[bidx.at[0]], selbuf)
        pltpu.sync_copy(selbuf, seltg_out.at[pl.ds(wid * BPT, BPT)])

    return k(out_sr, out_tg, rel_sr, rel_tg, hs, ts, rs, ht, tt, rt, srd, tgd)


def kernel(sr_data, tg_data, h_list_sr, h_list_tg, t_list_sr, t_list_tg,
           r_list_sr, r_list_tg, edge_index_sr, edge_index_tg,
           ent_emb_sr, ent_emb_tg, rel_emb_sr, rel_emb_tg,
           gat_W, gat_a_src, gat_a_dst):
    f32 = jnp.float32
    i32 = jnp.int32

    def prep_edges(ei):
        ei = ei.astype(i32)
        src = jnp.pad(ei[0], (0, EPAD - E)).reshape(EPAD // CHW, CHW)
        dst = jnp.pad(ei[1], (0, EPAD - E)).reshape(EPAD // CHW, CHW)
        return src, dst

    def prep_tlist(x):
        return jnp.pad(x.astype(i32), (0, TROWS * CHW - T)).reshape(
            TROWS, 1, CHW)

    def make_a(layer):
        a = jnp.zeros((DIM, DIM), f32)
        a = a.at[0:DH, 0].set(gat_a_src[layer, 0])
        a = a.at[0:DH, 1].set(gat_a_dst[layer, 0])
        a = a.at[DH:DIM, 2].set(gat_a_src[layer, 1])
        a = a.at[DH:DIM, 3].set(gat_a_dst[layer, 1])
        return a

    wc = [jnp.concatenate([gat_W[l, 0], gat_W[l, 1]], axis=1)
          for l in range(NLAYER)]
    ac = [make_a(l) for l in range(NLAYER)]
    z64 = jnp.zeros((NPAD, DH), f32)
    z16 = jnp.zeros((NPAD, 16), f32)

    def edge_phase(wh, evf, srcp, dstp):
        rawp, denp = _edge_sc(wh, evf[:, :16], srcp, dstp, z64, z16)
        return rawp[0], denp[0], rawp[1], denp[1]

    def gat(emb, srcp, dstp):
        emb_p = jnp.pad(emb.astype(f32), ((0, NPAD - N), (0, 0)))
        wh, evf = _proj(emb_p, wc[0], ac[0])
        parts = edge_phase(wh, evf, srcp, dstp)
        wh, evf = _combine_project(parts[0], parts[1], parts[2], parts[3],
                                   wc[1], ac[1])
        parts = edge_phase(wh, evf, srcp, dstp)
        return _combine_final(parts[0], parts[1], parts[2], parts[3])

    out_sr = gat(ent_emb_sr, *prep_edges(edge_index_sr))
    out_sr, emb_tg_seq = lax.optimization_barrier((out_sr, ent_emb_tg))
    out_tg = gat(emb_tg_seq, *prep_edges(edge_index_tg))

    transe, sel_sr, sel_tg = _post_sc(
        out_sr, out_tg, rel_emb_sr.astype(f32), rel_emb_tg.astype(f32),
        prep_tlist(h_list_sr), prep_tlist(t_list_sr), prep_tlist(r_list_sr),
        prep_tlist(h_list_tg), prep_tlist(t_list_tg), prep_tlist(r_list_tg),
        sr_data.astype(i32).reshape(NTILES, 1, BPT),
        tg_data.astype(i32).reshape(NTILES, 1, BPT),
    )
    return sel_sr, sel_tg, transe


# speedup vs baseline: 18.9958x; 2.0175x over previous
import functools

import jax
import jax.numpy as jnp
from jax import lax
from jax.experimental import pallas as pl
from jax.experimental.pallas import tpu as pltpu
from jax.experimental.pallas import tpu_sc as plsc

N = 10000
NPAD = 10112
DIM = 128
DH = 64
NHEADS = 2
NLAYER = 2
E = 320000
T = 100000
BATCH = 4096
ALPHA = 0.2

NTILES = 32
CHW = 128
NCH = 80
EPT = NCH * CHW
EPAD = NTILES * EPT
NPT = NPAD // 16

NFC = T // CHW
TREM = T - NFC * CHW
TITER = (NFC + NTILES - 1) // NTILES + 1
TROWS = NFC + 1
BPT = BATCH // NTILES

BM = 632


def _proj_body(x_ref, w_ref, a_ref, wh0_ref, wh1_ref, ev_ref):
    wh = jnp.dot(x_ref[...], w_ref[...], preferred_element_type=jnp.float32)
    wh0_ref[...] = wh[:, :DH]
    wh1_ref[...] = wh[:, DH:]
    ev_ref[...] = jnp.dot(wh, a_ref[...], preferred_element_type=jnp.float32)


_WH_OUT_SPECS = [
    pl.BlockSpec((BM, DH), lambda i: (i, 0)),
    pl.BlockSpec((BM, DH), lambda i: (i, 0)),
    pl.BlockSpec((BM, DIM), lambda i: (i, 0)),
]
_WH_OUT_SHAPE = [
    jax.ShapeDtypeStruct((NPAD, DH), jnp.float32),
    jax.ShapeDtypeStruct((NPAD, DH), jnp.float32),
    jax.ShapeDtypeStruct((NPAD, DIM), jnp.float32),
]


def _proj(x, wc, ac):
    return pl.pallas_call(
        _proj_body,
        grid=(NPAD // BM,),
        in_specs=[
            pl.BlockSpec((BM, DIM), lambda i: (i, 0)),
            pl.BlockSpec((DIM, DIM), lambda i: (0, 0)),
            pl.BlockSpec((DIM, DIM), lambda i: (0, 0)),
        ],
        out_specs=_WH_OUT_SPECS,
        out_shape=_WH_OUT_SHAPE,
    )(x, wc, ac)


def _normalize(r00, r01, r10, r11, d00, d01, d10, d11):
    raw0 = r00[...] + r01[...]
    raw1 = r10[...] + r11[...]
    den0 = d00[...] + d01[...]
    den1 = d10[...] + d11[...]
    rec0 = 1.0 / (den0[:, 0:1] + 1e-16)
    rec1 = 1.0 / (den1[:, 0:1] + 1e-16)
    return jnp.concatenate(
        [raw0 * jnp.broadcast_to(rec0, (BM, DH)),
         raw1 * jnp.broadcast_to(rec1, (BM, DH))], axis=1)


_NORM_SPECS = (
    [pl.BlockSpec((BM, DH), lambda i: (i, 0))] * 4
    + [pl.BlockSpec((BM, 16), lambda i: (i, 0))] * 4
)


def _combine_project_body(r00, r01, r10, r11, d00, d01, d10, d11,
                          w_ref, a_ref, wh0_ref, wh1_ref, ev_ref):
    h = _normalize(r00, r01, r10, r11, d00, d01, d10, d11)
    h = jnp.where(h > 0, h, jnp.exp(h) - 1.0)
    wh = jnp.dot(h, w_ref[...], preferred_element_type=jnp.float32)
    wh0_ref[...] = wh[:, :DH]
    wh1_ref[...] = wh[:, DH:]
    ev_ref[...] = jnp.dot(wh, a_ref[...], preferred_element_type=jnp.float32)


def _combine_project(raw0, den0, raw1, den1, wc, ac):
    return pl.pallas_call(
        _combine_project_body,
        grid=(NPAD // BM,),
        in_specs=_NORM_SPECS + [
            pl.BlockSpec((DIM, DIM), lambda i: (0, 0)),
            pl.BlockSpec((DIM, DIM), lambda i: (0, 0)),
        ],
        out_specs=_WH_OUT_SPECS,
        out_shape=_WH_OUT_SHAPE,
    )(raw0[0], raw0[1], raw1[0], raw1[1],
      den0[0], den0[1], den1[0], den1[1], wc, ac)


def _combine_final_body(r00, r01, r10, r11, d00, d01, d10, d11, o_ref):
    o_ref[...] = _normalize(r00, r01, r10, r11, d00, d01, d10, d11)


def _combine_final(raw0, den0, raw1, den1):
    return pl.pallas_call(
        _combine_final_body,
        grid=(NPAD // BM,),
        in_specs=_NORM_SPECS,
        out_specs=pl.BlockSpec((BM, DIM), lambda i: (i, 0)),
        out_shape=jax.ShapeDtypeStruct((NPAD, DIM), jnp.float32),
    )(raw0[0], raw0[1], raw1[0], raw1[1],
      den0[0], den0[1], den1[0], den1[1])


def _edge_sc(wh0, wh1, ev16, srcp, dstp, z64, z16):
    mesh = plsc.VectorSubcoreMesh(core_axis_name="c", subcore_axis_name="s")

    @functools.partial(
        pl.kernel,
        out_type=(
            jax.ShapeDtypeStruct((2, 2, NPAD, DH), jnp.float32),
            jax.ShapeDtypeStruct((2, 2, NPAD, 16), jnp.float32),
        ),
        mesh=mesh,
        scratch_types=[
            pltpu.VMEM((2, CHW, 16), jnp.float32),
            pltpu.VMEM((2, CHW, 16), jnp.float32),
            pltpu.VMEM((NCH, CHW), jnp.int32),
            pltpu.VMEM((NCH, CHW), jnp.int32),
            pltpu.VMEM((2, CHW, DH), jnp.float32),
            pltpu.VMEM((2, CHW, 16), jnp.float32),
            pltpu.VMEM_SHARED((NPAD, DH), jnp.float32),
            pltpu.VMEM_SHARED((NPAD, 16), jnp.float32),
            pltpu.VMEM_SHARED((NPAD, 16), jnp.float32),
            pltpu.SemaphoreType.DMA((10,)),
        ],
        compiler_params=pltpu.CompilerParams(use_tc_tiling_on_sc=False),
    )
    def k(wh0_hbm, wh1_hbm, ev_hbm, src_hbm, dst_hbm, z64_hbm, z16_hbm,
          raw_out, den_out, sbuf, dbuf, sidx, didx, rows, stag,
          spraw, spden, spev, sems):
        cid = lax.axis_index("c")
        sid = lax.axis_index("s")
        wid = cid * 16 + sid
        r0 = sid * NPT
        pltpu.sync_copy(ev_hbm.at[pl.ds(r0, NPT)], spev.at[pl.ds(r0, NPT)])
        pltpu.sync_copy(src_hbm.at[pl.ds(wid * NCH, NCH)], sidx)
        pltpu.sync_copy(dst_hbm.at[pl.ds(wid * NCH, NCH)], didx)

        base_eid = wid * EPT
        iota = lax.iota(jnp.int32, 16)
        m_lane0 = iota == 0

        def g_descs(j, p, wh_hbm):
            return [
                (wh_hbm.at[sidx.at[j]], rows.at[p], sems.at[p * 5 + 0]),
                (spev.at[sidx.at[j]], sbuf.at[p], sems.at[p * 5 + 1]),
                (spev.at[didx.at[j]], dbuf.at[p], sems.at[p * 5 + 2]),
            ]

        def issue_gathers(j, p, wh_hbm):
            for s, d, m in g_descs(j, p, wh_hbm):
                pltpu.async_copy(s, d, m)

        def wait_gathers(j, p, wh_hbm):
            for s, d, m in g_descs(j, p, wh_hbm):
                pltpu.make_async_copy(s, d, m).wait()

        def issue_scatters(j, p):
            pltpu.async_copy(rows.at[p], spraw.at[didx.at[j]],
                             sems.at[p * 5 + 3], add=True)
            pltpu.async_copy(stag.at[p], spden.at[didx.at[j]],
                             sems.at[p * 5 + 4], add=True)

        def wait_scatters(j, p):
            pltpu.make_async_copy(rows.at[p], spraw.at[didx.at[j]],
                                  sems.at[p * 5 + 3]).wait()
            pltpu.make_async_copy(stag.at[p], spden.at[didx.at[j]],
                                  sems.at[p * 5 + 4]).wait()

        def compute(j, p, head):
            base_j = base_eid + j * CHW

            def edge(kk, c):
                vs = sbuf[p, kk, pl.ds(0, 16)]
                vd = dbuf[p, kk, pl.ds(0, 16)]
                e = vs[2 * head] + vd[2 * head + 1]
                e = jnp.maximum(e, ALPHA * e)
                ev = jnp.where(m_lane0, e, 0.0)
                exv = jnp.exp(ev)
                exv = jnp.where(m_lane0, exv, 0.0)
                valid = (base_j + kk) < E
                exv = jnp.where(valid, exv, 0.0)
                stag[p, kk, pl.ds(0, 16)] = exv
                x = exv[0]
                for cc in range(DH // 16):
                    sl = pl.ds(cc * 16, 16)
                    rows[p, kk, sl] = rows[p, kk, sl] * x
                return c

            lax.fori_loop(0, CHW, edge, 0)

        for head in range(NHEADS):
            wh_hbm = wh0_hbm if head == 0 else wh1_hbm

            def zcopy(q, c):
                pltpu.sync_copy(z64_hbm, spraw.at[pl.ds(r0 + q * 79, 79)])
                pltpu.sync_copy(z16_hbm, spden.at[pl.ds(r0 + q * 79, 79)])
                return c

            lax.fori_loop(0, NPT // 79, zcopy, 0)
            plsc.subcore_barrier()

            def prol(q, c, wh_hbm=wh_hbm):
                issue_gathers(q, q, wh_hbm)
                return c

            lax.fori_loop(0, 2, prol, 0)

            def body(j, c, head=head, wh_hbm=wh_hbm):
                p = lax.rem(j, 2)
                wait_gathers(j, p, wh_hbm)
                compute(j, p, head)
                issue_scatters(j, p)
                wait_scatters(j, p)

                @pl.when(j < NCH - 2)
                def _():
                    issue_gathers(j + 2, p, wh_hbm)

                return c

            lax.fori_loop(0, NCH, body, 0)
            plsc.subcore_barrier()
            pltpu.sync_copy(spraw.at[pl.ds(r0, NPT)],
                            raw_out.at[head, cid, pl.ds(r0, NPT)])
            pltpu.sync_copy(spden.at[pl.ds(r0, NPT)],
                            den_out.at[head, cid, pl.ds(r0, NPT)])

    return k(wh0, wh1, ev16, srcp, dstp, z64, z16)


def _post_sc(out_sr, out_tg, rel_sr, rel_tg,
             hs, ts, rs, ht, tt, rt, srd, tgd):
    mesh = plsc.VectorSubcoreMesh(core_axis_name="c", subcore_axis_name="s")

    @functools.partial(
        pl.kernel,
        out_type=(
            jax.ShapeDtypeStruct((2 * T, DIM), jnp.float32),
            jax.ShapeDtypeStruct((BATCH, DIM), jnp.float32),
            jax.ShapeDtypeStruct((BATCH, DIM), jnp.float32),
        ),
        mesh=mesh,
        scratch_types=[
            pltpu.VMEM((1, CHW), jnp.int32),
            pltpu.VMEM((1, CHW), jnp.int32),
            pltpu.VMEM((1, CHW), jnp.int32),
            pltpu.VMEM((CHW, DIM), jnp.float32),
            pltpu.VMEM((CHW, DIM), jnp.float32),
            pltpu.VMEM((CHW, DIM), jnp.float32),
            pltpu.VMEM((1, BPT), jnp.int32),
            pltpu.VMEM((BPT, DIM), jnp.float32),
        ],
    )
    def k(osr_hbm, otg_hbm, rsr_hbm, rtg_hbm,
          hs_hbm, ts_hbm, rs_hbm, ht_hbm, tt_hbm, rt_hbm, srd_hbm, tgd_hbm,
          tr_out, selsr_out, seltg_out,
          hidx, tidx, ridx, hbuf, tbuf, rbuf, bidx, selbuf):
        cid = lax.axis_index("c")
        sid = lax.axis_index("s")
        wid = cid * 16 + sid

        def combine(nrows):
            def row(kk, cc):
                for c8 in range(DIM // 16):
                    sl = pl.ds(c8 * 16, 16)
                    hbuf[kk, sl] = hbuf[kk, sl] + rbuf[kk, sl] - tbuf[kk, sl]
                return cc
            lax.fori_loop(0, nrows, row, 0)

        for graph in range(2):
            ent = osr_hbm if graph == 0 else otg_hbm
            rel = rsr_hbm if graph == 0 else rtg_hbm
            hl = hs_hbm if graph == 0 else ht_hbm
            tl = ts_hbm if graph == 0 else tt_hbm
            rl = rs_hbm if graph == 0 else rt_hbm
            obase = graph * T

            def tchunk(i, c, ent=ent, rel=rel, hl=hl, tl=tl, rl=rl,
                       obase=obase):
                cix = wid + i * NTILES

                @pl.when(cix < NFC)
                def _():
                    pltpu.sync_copy(hl.at[cix], hidx)
                    pltpu.sync_copy(tl.at[cix], tidx)
                    pltpu.sync_copy(rl.at[cix], ridx)
                    pltpu.sync_copy(ent.at[hidx.at[0]], hbuf)
                    pltpu.sync_copy(ent.at[tidx.at[0]], tbuf)
                    pltpu.sync_copy(rel.at[ridx.at[0]], rbuf)
                    combine(CHW)
                    pltpu.sync_copy(
                        hbuf, tr_out.at[pl.ds(obase + cix * CHW, CHW)])
                return c

            lax.fori_loop(0, TITER, tchunk, 0)

            @pl.when(wid == NFC % NTILES)
            def _(ent=ent, rel=rel, hl=hl, tl=tl, rl=rl, obase=obase):
                pltpu.sync_copy(hl.at[NFC], hidx)
                pltpu.sync_copy(tl.at[NFC], tidx)
                pltpu.sync_copy(rl.at[NFC], ridx)
                pltpu.sync_copy(ent.at[hidx.at[0, pl.ds(0, TREM)]],
                                hbuf.at[pl.ds(0, TREM)])
                pltpu.sync_copy(ent.at[tidx.at[0, pl.ds(0, TREM)]],
                                tbuf.at[pl.ds(0, TREM)])
                pltpu.sync_copy(rel.at[ridx.at[0, pl.ds(0, TREM)]],
                                rbuf.at[pl.ds(0, TREM)])
                combine(TREM)
                pltpu.sync_copy(hbuf.at[pl.ds(0, TREM)],
                                tr_out.at[pl.ds(obase + NFC * CHW, TREM)])

        pltpu.sync_copy(srd_hbm.at[wid], bidx)
        pltpu.sync_copy(osr_hbm.at[bidx.at[0]], selbuf)
        pltpu.sync_copy(selbuf, selsr_out.at[pl.ds(wid * BPT, BPT)])
        pltpu.sync_copy(tgd_hbm.at[wid], bidx)
        pltpu.sync_copy(otg_hbm.at[bidx.at[0]], selbuf)
        pltpu.sync_copy(selbuf, seltg_out.at[pl.ds(wid * BPT, BPT)])

    return k(out_sr, out_tg, rel_sr, rel_tg, hs, ts, rs, ht, tt, rt, srd, tgd)


def kernel(sr_data, tg_data, h_list_sr, h_list_tg, t_list_sr, t_list_tg,
           r_list_sr, r_list_tg, edge_index_sr, edge_index_tg,
           ent_emb_sr, ent_emb_tg, rel_emb_sr, rel_emb_tg,
           gat_W, gat_a_src, gat_a_dst):
    f32 = jnp.float32
    i32 = jnp.int32

    def prep_edges(ei):
        ei = ei.astype(i32)
        src = jnp.pad(ei[0], (0, EPAD - E)).reshape(EPAD // CHW, CHW)
        dst = jnp.pad(ei[1], (0, EPAD - E)).reshape(EPAD // CHW, CHW)
        return src, dst

    def prep_tlist(x):
        return jnp.pad(x.astype(i32), (0, TROWS * CHW - T)).reshape(
            TROWS, 1, CHW)

    def make_a(layer):
        a = jnp.zeros((DIM, DIM), f32)
        a = a.at[0:DH, 0].set(gat_a_src[layer, 0])
        a = a.at[0:DH, 1].set(gat_a_dst[layer, 0])
        a = a.at[DH:DIM, 2].set(gat_a_src[layer, 1])
        a = a.at[DH:DIM, 3].set(gat_a_dst[layer, 1])
        return a

    wc = [jnp.concatenate([gat_W[l, 0], gat_W[l, 1]], axis=1)
          for l in range(NLAYER)]
    ac = [make_a(l) for l in range(NLAYER)]
    z64 = jnp.zeros((79, DH), f32)
    z16 = jnp.zeros((79, 16), f32)

    def edge_phase(wh0, wh1, evf, srcp, dstp):
        rawp, denp = _edge_sc(wh0, wh1, evf[:, :16], srcp, dstp, z64, z16)
        return rawp[0], denp[0], rawp[1], denp[1]

    def gat(emb, srcp, dstp):
        emb_p = jnp.pad(emb.astype(f32), ((0, NPAD - N), (0, 0)))
        wh0, wh1, evf = _proj(emb_p, wc[0], ac[0])
        parts = edge_phase(wh0, wh1, evf, srcp, dstp)
        wh0, wh1, evf = _combine_project(parts[0], parts[1], parts[2],
                                         parts[3], wc[1], ac[1])
        parts = edge_phase(wh0, wh1, evf, srcp, dstp)
        return _combine_final(parts[0], parts[1], parts[2], parts[3])

    out_sr = gat(ent_emb_sr, *prep_edges(edge_index_sr))
    out_sr, emb_tg_seq = lax.optimization_barrier((out_sr, ent_emb_tg))
    out_tg = gat(emb_tg_seq, *prep_edges(edge_index_tg))

    transe, sel_sr, sel_tg = _post_sc(
        out_sr, out_tg, rel_emb_sr.astype(f32), rel_emb_tg.astype(f32),
        prep_tlist(h_list_sr), prep_tlist(t_list_sr), prep_tlist(r_list_sr),
        prep_tlist(h_list_tg), prep_tlist(t_list_tg), prep_tlist(r_list_tg),
        sr_data.astype(i32).reshape(NTILES, 1, BPT),
        tg_data.astype(i32).reshape(NTILES, 1, BPT),
    )
    return sel_sr, sel_tg, transe


# speedup vs baseline: 29.0856x; 1.5312x over previous
import functools

import jax
import jax.numpy as jnp
from jax import lax
from jax.experimental import pallas as pl
from jax.experimental.pallas import tpu as pltpu
from jax.experimental.pallas import tpu_sc as plsc

N = 10000
NPAD = 10112
DIM = 128
DH = 64
NHEADS = 2
NLAYER = 2
E = 320000
T = 100000
BATCH = 4096
ALPHA = 0.2

NTILES = 32
CHW = 128
NCH = 80
EPT = NCH * CHW
EPAD = NTILES * EPT
NPT = NPAD // 16

NFC = T // CHW
TREM = T - NFC * CHW
TITER = (NFC + NTILES - 1) // NTILES + 1
TROWS = NFC + 1
BPT = BATCH // NTILES

BM = 632


def _proj_body(x_ref, w_ref, a_ref, wh0_ref, wh1_ref, ev_ref):
    wh = jnp.dot(x_ref[...], w_ref[...], preferred_element_type=jnp.float32)
    wh0_ref[...] = wh[:, :DH]
    wh1_ref[...] = wh[:, DH:]
    ev_ref[...] = jnp.dot(wh, a_ref[...], preferred_element_type=jnp.float32)


_WH_OUT_SPECS = [
    pl.BlockSpec((BM, DH), lambda i: (i, 0)),
    pl.BlockSpec((BM, DH), lambda i: (i, 0)),
    pl.BlockSpec((BM, DIM), lambda i: (i, 0)),
]
_WH_OUT_SHAPE = [
    jax.ShapeDtypeStruct((NPAD, DH), jnp.float32),
    jax.ShapeDtypeStruct((NPAD, DH), jnp.float32),
    jax.ShapeDtypeStruct((NPAD, DIM), jnp.float32),
]


def _proj(x, wc, ac):
    return pl.pallas_call(
        _proj_body,
        grid=(NPAD // BM,),
        in_specs=[
            pl.BlockSpec((BM, DIM), lambda i: (i, 0)),
            pl.BlockSpec((DIM, DIM), lambda i: (0, 0)),
            pl.BlockSpec((DIM, DIM), lambda i: (0, 0)),
        ],
        out_specs=_WH_OUT_SPECS,
        out_shape=_WH_OUT_SHAPE,
    )(x, wc, ac)


def _normalize(r00, r01, r10, r11, d00, d01, d10, d11):
    raw0 = r00[...] + r01[...]
    raw1 = r10[...] + r11[...]
    den0 = d00[...] + d01[...]
    den1 = d10[...] + d11[...]
    rec0 = 1.0 / (den0 + 1e-16)
    rec1 = 1.0 / (den1 + 1e-16)
    return jnp.concatenate(
        [raw0 * jnp.broadcast_to(rec0, (BM, DH)),
         raw1 * jnp.broadcast_to(rec1, (BM, DH))], axis=1)


_NORM_SPECS = (
    [pl.BlockSpec((BM, DH), lambda i: (i, 0))] * 4
    + [pl.BlockSpec((BM, 1), lambda i: (i, 0))] * 4
)


def _combine_project_body(r00, r01, r10, r11, d00, d01, d10, d11,
                          w_ref, a_ref, wh0_ref, wh1_ref, ev_ref):
    h = _normalize(r00, r01, r10, r11, d00, d01, d10, d11)
    h = jnp.where(h > 0, h, jnp.exp(h) - 1.0)
    wh = jnp.dot(h, w_ref[...], preferred_element_type=jnp.float32)
    wh0_ref[...] = wh[:, :DH]
    wh1_ref[...] = wh[:, DH:]
    ev_ref[...] = jnp.dot(wh, a_ref[...], preferred_element_type=jnp.float32)


def _combine_project(raw0, den0, raw1, den1, wc, ac):
    return pl.pallas_call(
        _combine_project_body,
        grid=(NPAD // BM,),
        in_specs=_NORM_SPECS + [
            pl.BlockSpec((DIM, DIM), lambda i: (0, 0)),
            pl.BlockSpec((DIM, DIM), lambda i: (0, 0)),
        ],
        out_specs=_WH_OUT_SPECS,
        out_shape=_WH_OUT_SHAPE,
    )(raw0[0], raw0[1], raw1[0], raw1[1],
      den0[0], den0[1], den1[0], den1[1], wc, ac)


def _combine_final_body(r00, r01, r10, r11, d00, d01, d10, d11, o_ref):
    o_ref[...] = _normalize(r00, r01, r10, r11, d00, d01, d10, d11)


def _combine_final(raw0, den0, raw1, den1):
    return pl.pallas_call(
        _combine_final_body,
        grid=(NPAD // BM,),
        in_specs=_NORM_SPECS,
        out_specs=pl.BlockSpec((BM, DIM), lambda i: (i, 0)),
        out_shape=jax.ShapeDtypeStruct((NPAD, DIM), jnp.float32),
    )(raw0[0], raw0[1], raw1[0], raw1[1],
      den0[0], den0[1], den1[0], den1[1])


def _edge_sc(wh0, wh1, evt, srcp, dstp, z64, z1):
    mesh = plsc.VectorSubcoreMesh(core_axis_name="c", subcore_axis_name="s")

    @functools.partial(
        pl.kernel,
        out_type=(
            jax.ShapeDtypeStruct((2, 2, NPAD, DH), jnp.float32),
            jax.ShapeDtypeStruct((2, 2, NPAD), jnp.float32),
        ),
        mesh=mesh,
        scratch_types=[
            pltpu.VMEM((2, CHW), jnp.float32),
            pltpu.VMEM((2, CHW), jnp.float32),
            pltpu.VMEM((NCH, CHW), jnp.int32),
            pltpu.VMEM((NCH, CHW), jnp.int32),
            pltpu.VMEM((2, CHW, DH), jnp.float32),
            pltpu.VMEM((2, CHW, DH), jnp.float32),
            pltpu.VMEM((2, CHW), jnp.float32),
            pltpu.VMEM_SHARED((NPAD, DH), jnp.float32),
            pltpu.VMEM_SHARED((NPAD,), jnp.float32),
            pltpu.VMEM_SHARED((NPAD,), jnp.float32),
            pltpu.VMEM_SHARED((NPAD,), jnp.float32),
            pltpu.VMEM_SHARED((NPAD,), jnp.float32),
            pltpu.VMEM_SHARED((NPAD,), jnp.float32),
            pltpu.SemaphoreType.DMA((10,)),
        ],
        compiler_params=pltpu.CompilerParams(use_tc_tiling_on_sc=False),
    )
    def k(wh0_hbm, wh1_hbm, evt_hbm, src_hbm, dst_hbm, z64_hbm, z1_hbm,
          raw_out, den_out, sbuf, dbuf, sidx, didx, rows, hrows, exbuf,
          spraw, spden, spes0, sped0, spes1, sped1, sems):
        cid = lax.axis_index("c")
        sid = lax.axis_index("s")
        wid = cid * 16 + sid
        r0 = sid * NPT
        for c, spt in enumerate((spes0, sped0, spes1, sped1)):
            pltpu.sync_copy(evt_hbm.at[c, pl.ds(r0, NPT)],
                            spt.at[pl.ds(r0, NPT)])
        pltpu.sync_copy(src_hbm.at[pl.ds(wid * NCH, NCH)], sidx)
        pltpu.sync_copy(dst_hbm.at[pl.ds(wid * NCH, NCH)], didx)

        base_eid = wid * EPT
        iota = lax.iota(jnp.int32, 16)

        def g_descs(j, p, wh_hbm, spes, sped):
            return [
                (wh_hbm.at[sidx.at[j]], rows.at[p], sems.at[p * 5 + 0]),
                (spes.at[sidx.at[j]], sbuf.at[p], sems.at[p * 5 + 1]),
                (sped.at[didx.at[j]], dbuf.at[p], sems.at[p * 5 + 2]),
            ]

        def issue_gathers(j, p, tabs):
            for s, d, m in g_descs(j, p, *tabs):
                pltpu.async_copy(s, d, m)

        def wait_gathers(j, p, tabs):
            for s, d, m in g_descs(j, p, *tabs):
                pltpu.make_async_copy(s, d, m).wait()

        def issue_scatters(j, p):
            pltpu.async_copy(hrows.at[p], spraw.at[didx.at[j]],
                             sems.at[p * 5 + 3], add=True)
            pltpu.async_copy(exbuf.at[p], spden.at[didx.at[j]],
                             sems.at[p * 5 + 4], add=True)

        def wait_scatters(j, p):
            pltpu.make_async_copy(hrows.at[p], spraw.at[didx.at[j]],
                                  sems.at[p * 5 + 3]).wait()
            pltpu.make_async_copy(exbuf.at[p], spden.at[didx.at[j]],
                                  sems.at[p * 5 + 4]).wait()

        def compute(j, p):
            base_j = base_eid + j * CHW
            for g in range(CHW // 16):
                sl16 = pl.ds(g * 16, 16)
                e = sbuf[p, sl16] + dbuf[p, sl16]
                e = jnp.maximum(e, ALPHA * e)
                exv = jnp.exp(e)
                eid = base_j + g * 16 + iota
                exv = jnp.where(eid < E, exv, 0.0)
                exbuf[p, sl16] = exv
                for i in range(16):
                    kk = g * 16 + i
                    x = exv[i]
                    for cc in range(DH // 16):
                        sl = pl.ds(cc * 16, 16)
                        hrows[p, kk, sl] = rows[p, kk, sl] * x

        for head in range(NHEADS):
            wh_hbm = wh0_hbm if head == 0 else wh1_hbm
            tabs = ((wh0_hbm, spes0, sped0), (wh1_hbm, spes1, sped1))[head]
            pltpu.sync_copy(z64_hbm, spraw.at[pl.ds(r0, NPT)])
            pltpu.sync_copy(z1_hbm, spden.at[pl.ds(r0, NPT)])
            plsc.subcore_barrier()

            def prol(q, c, tabs=tabs):
                issue_gathers(q, q, tabs)
                return c

            lax.fori_loop(0, 2, prol, 0)

            def body(j, c, tabs=tabs):
                p = lax.rem(j, 2)
                wait_gathers(j, p, tabs)

                @pl.when(j >= 2)
                def _():
                    wait_scatters(j, p)

                compute(j, p)
                issue_scatters(j, p)

                @pl.when(j < NCH - 2)
                def _():
                    issue_gathers(j + 2, p, tabs)

                return c

            lax.fori_loop(0, NCH, body, 0)

            def epi(j, c):
                wait_scatters(j, lax.rem(j, 2))
                return c

            lax.fori_loop(NCH - 2, NCH, epi, 0)
            plsc.subcore_barrier()
            pltpu.sync_copy(spraw.at[pl.ds(r0, NPT)],
                            raw_out.at[head, cid, pl.ds(r0, NPT)])
            pltpu.sync_copy(spden.at[pl.ds(r0, NPT)],
                            den_out.at[head, cid, pl.ds(r0, NPT)])

    return k(wh0, wh1, evt, srcp, dstp, z64, z1)


def _post_sc(out_sr, out_tg, rel_sr, rel_tg,
             hs, ts, rs, ht, tt, rt, srd, tgd):
    mesh = plsc.VectorSubcoreMesh(core_axis_name="c", subcore_axis_name="s")

    @functools.partial(
        pl.kernel,
        out_type=(
            jax.ShapeDtypeStruct((2 * T, DIM), jnp.float32),
            jax.ShapeDtypeStruct((BATCH, DIM), jnp.float32),
            jax.ShapeDtypeStruct((BATCH, DIM), jnp.float32),
        ),
        mesh=mesh,
        scratch_types=[
            pltpu.VMEM((1, CHW), jnp.int32),
            pltpu.VMEM((1, CHW), jnp.int32),
            pltpu.VMEM((1, CHW), jnp.int32),
            pltpu.VMEM((CHW, DIM), jnp.float32),
            pltpu.VMEM((CHW, DIM), jnp.float32),
            pltpu.VMEM((CHW, DIM), jnp.float32),
            pltpu.VMEM((1, BPT), jnp.int32),
            pltpu.VMEM((BPT, DIM), jnp.float32),
        ],
    )
    def k(osr_hbm, otg_hbm, rsr_hbm, rtg_hbm,
          hs_hbm, ts_hbm, rs_hbm, ht_hbm, tt_hbm, rt_hbm, srd_hbm, tgd_hbm,
          tr_out, selsr_out, seltg_out,
          hidx, tidx, ridx, hbuf, tbuf, rbuf, bidx, selbuf):
        cid = lax.axis_index("c")
        sid = lax.axis_index("s")
        wid = cid * 16 + sid

        def combine(nrows):
            def row(kk, cc):
                for c8 in range(DIM // 16):
                    sl = pl.ds(c8 * 16, 16)
                    hbuf[kk, sl] = hbuf[kk, sl] + rbuf[kk, sl] - tbuf[kk, sl]
                return cc
            lax.fori_loop(0, nrows, row, 0)

        for graph in range(2):
            ent = osr_hbm if graph == 0 else otg_hbm
            rel = rsr_hbm if graph == 0 else rtg_hbm
            hl = hs_hbm if graph == 0 else ht_hbm
            tl = ts_hbm if graph == 0 else tt_hbm
            rl = rs_hbm if graph == 0 else rt_hbm
            obase = graph * T

            def tchunk(i, c, ent=ent, rel=rel, hl=hl, tl=tl, rl=rl,
                       obase=obase):
                cix = wid + i * NTILES

                @pl.when(cix < NFC)
                def _():
                    pltpu.sync_copy(hl.at[cix], hidx)
                    pltpu.sync_copy(tl.at[cix], tidx)
                    pltpu.sync_copy(rl.at[cix], ridx)
                    pltpu.sync_copy(ent.at[hidx.at[0]], hbuf)
                    pltpu.sync_copy(ent.at[tidx.at[0]], tbuf)
                    pltpu.sync_copy(rel.at[ridx.at[0]], rbuf)
                    combine(CHW)
                    pltpu.sync_copy(
                        hbuf, tr_out.at[pl.ds(obase + cix * CHW, CHW)])
                return c

            lax.fori_loop(0, TITER, tchunk, 0)

            @pl.when(wid == NFC % NTILES)
            def _(ent=ent, rel=rel, hl=hl, tl=tl, rl=rl, obase=obase):
                pltpu.sync_copy(hl.at[NFC], hidx)
                pltpu.sync_copy(tl.at[NFC], tidx)
                pltpu.sync_copy(rl.at[NFC], ridx)
                pltpu.sync_copy(ent.at[hidx.at[0, pl.ds(0, TREM)]],
                                hbuf.at[pl.ds(0, TREM)])
                pltpu.sync_copy(ent.at[tidx.at[0, pl.ds(0, TREM)]],
                                tbuf.at[pl.ds(0, TREM)])
                pltpu.sync_copy(rel.at[ridx.at[0, pl.ds(0, TREM)]],
                                rbuf.at[pl.ds(0, TREM)])
                combine(TREM)
                pltpu.sync_copy(hbuf.at[pl.ds(0, TREM)],
                                tr_out.at[pl.ds(obase + NFC * CHW, TREM)])

        pltpu.sync_copy(srd_hbm.at[wid], bidx)
        pltpu.sync_copy(osr_hbm.at[bidx.at[0]], selbuf)
        pltpu.sync_copy(selbuf, selsr_out.at[pl.ds(wid * BPT, BPT)])
        pltpu.sync_copy(tgd_hbm.at[wid], bidx)
        pltpu.sync_copy(otg_hbm.at[bidx.at[0]], selbuf)
        pltpu.sync_copy(selbuf, seltg_out.at[pl.ds(wid * BPT, BPT)])

    return k(out_sr, out_tg, rel_sr, rel_tg, hs, ts, rs, ht, tt, rt, srd, tgd)


def kernel(sr_data, tg_data, h_list_sr, h_list_tg, t_list_sr, t_list_tg,
           r_list_sr, r_list_tg, edge_index_sr, edge_index_tg,
           ent_emb_sr, ent_emb_tg, rel_emb_sr, rel_emb_tg,
           gat_W, gat_a_src, gat_a_dst):
    f32 = jnp.float32
    i32 = jnp.int32

    def prep_edges(ei):
        ei = ei.astype(i32)
        src = jnp.pad(ei[0], (0, EPAD - E)).reshape(EPAD // CHW, CHW)
        dst = jnp.pad(ei[1], (0, EPAD - E)).reshape(EPAD // CHW, CHW)
        return src, dst

    def prep_tlist(x):
        return jnp.pad(x.astype(i32), (0, TROWS * CHW - T)).reshape(
            TROWS, 1, CHW)

    def make_a(layer):
        a = jnp.zeros((DIM, DIM), f32)
        a = a.at[0:DH, 0].set(gat_a_src[layer, 0])
        a = a.at[0:DH, 1].set(gat_a_dst[layer, 0])
        a = a.at[DH:DIM, 2].set(gat_a_src[layer, 1])
        a = a.at[DH:DIM, 3].set(gat_a_dst[layer, 1])
        return a

    wc = [jnp.concatenate([gat_W[l, 0], gat_W[l, 1]], axis=1)
          for l in range(NLAYER)]
    ac = [make_a(l) for l in range(NLAYER)]
    z64 = jnp.zeros((NPT, DH), f32)
    z1 = jnp.zeros((NPT,), f32)

    def edge_phase(wh0, wh1, evf, srcp, dstp):
        evt = jnp.transpose(evf[:, :4])
        rawp, denp = _edge_sc(wh0, wh1, evt, srcp, dstp, z64, z1)
        dd = denp[..., None]
        return rawp[0], dd[0], rawp[1], dd[1]

    def gat(emb, srcp, dstp):
        emb_p = jnp.pad(emb.astype(f32), ((0, NPAD - N), (0, 0)))
        wh0, wh1, evf = _proj(emb_p, wc[0], ac[0])
        parts = edge_phase(wh0, wh1, evf, srcp, dstp)
        wh0, wh1, evf = _combine_project(parts[0], parts[1], parts[2],
                                         parts[3], wc[1], ac[1])
        parts = edge_phase(wh0, wh1, evf, srcp, dstp)
        return _combine_final(parts[0], parts[1], parts[2], parts[3])

    out_sr = gat(ent_emb_sr, *prep_edges(edge_index_sr))
    out_sr, emb_tg_seq = lax.optimization_barrier((out_sr, ent_emb_tg))
    out_tg = gat(emb_tg_seq, *prep_edges(edge_index_tg))

    transe, sel_sr, sel_tg = _post_sc(
        out_sr, out_tg, rel_emb_sr.astype(f32), rel_emb_tg.astype(f32),
        prep_tlist(h_list_sr), prep_tlist(t_list_sr), prep_tlist(r_list_sr),
        prep_tlist(h_list_tg), prep_tlist(t_list_tg), prep_tlist(r_list_tg),
        sr_data.astype(i32).reshape(NTILES, 1, BPT),
        tg_data.astype(i32).reshape(NTILES, 1, BPT),
    )
    return sel_sr, sel_tg, transe
